# baseline (speedup 1.0000x reference)
import functools

import jax
import jax.numpy as jnp
from jax import lax
from jax.experimental import pallas as pl
from jax.experimental.pallas import tpu as pltpu
from jax.experimental.pallas import tpu_sc as plsc

_L = 16
_D = 64
_SEQ = 200
_NCH = _D // _L


def _body(idx_hbm, emb_hbm, w_hbm, b_hbm, out_hbm,
          idx_v, rows_v, w_v, b_v, out_v, tmp_v, sem):
    c = lax.axis_index("c")
    s = lax.axis_index("s")

    @pl.when(jnp.logical_and(c == 0, s == 0))
    def _():
        pltpu.sync_copy(idx_hbm, idx_v)
        g1 = pltpu.async_copy(emb_hbm.at[idx_v.at[pl.ds(0, 128)]],
                              rows_v.at[pl.ds(0, 128)], sem)
        g2 = pltpu.async_copy(emb_hbm.at[idx_v.at[pl.ds(128, 72)]],
                              rows_v.at[pl.ds(128, 72)], sem)
        pltpu.sync_copy(w_hbm, w_v)
        pltpu.sync_copy(b_hbm, b_v)
        g1.wait()
        g2.wait()

        init = tuple(rows_v[0, pl.ds(d * _L, _L)] for d in range(_NCH))

        def step(r, accs):
            return tuple(jnp.maximum(a, rows_v[r, pl.ds(d * _L, _L)])
                         for d, a in enumerate(accs))

        accs = lax.fori_loop(1, _SEQ, step, init)

        lane = lax.iota(jnp.int32, _L)
        vec = jnp.zeros((_L,), jnp.float32)
        for j in range(2):
            psum = jnp.zeros((_L,), jnp.float32)
            for d in range(_NCH):
                psum = psum + accs[d] * w_v[j, pl.ds(d * _L, _L)]
            t = psum[0]
            for i in range(1, _L):
                t = t + psum[i]
            vec = jnp.where(lane == j, t, vec)
        out_v[...] = vec + b_v[...]
        pltpu.sync_copy(out_v, out_hbm)


_mesh = plsc.VectorSubcoreMesh(core_axis_name="c", subcore_axis_name="s",
                               num_cores=2, num_subcores=16)

_call = functools.partial(
    pl.kernel,
    out_type=jax.ShapeDtypeStruct((_L,), jnp.float32),
    mesh=_mesh,
    compiler_params=pltpu.CompilerParams(use_tc_tiling_on_sc=False),
    scratch_types=[
        pltpu.VMEM((_SEQ,), jnp.int32),
        pltpu.VMEM((_SEQ, _D), jnp.float32),
        pltpu.VMEM((2, _D), jnp.float32),
        pltpu.VMEM((_L,), jnp.float32),
        pltpu.VMEM((_L,), jnp.float32),
        pltpu.VMEM((_L,), jnp.float32),
        pltpu.SemaphoreType.DMA,
    ],
)(_body)


@jax.jit
def kernel(x, emb, W, b):
    idx = x.reshape(-1).astype(jnp.int32)
    b16 = jnp.pad(b.astype(jnp.float32), (0, _L - b.shape[0]))
    out16 = _call(idx, emb, W, b16)
    return out16[:2].reshape(1, 2)


# speedup vs baseline: 1.7040x; 1.7040x over previous
import functools

import jax
import jax.numpy as jnp
from jax import lax
from jax.experimental import pallas as pl
from jax.experimental.pallas import tpu as pltpu
from jax.experimental.pallas import tpu_sc as plsc

_L = 16
_D = 64
_SEQ = 200
_NCH = _D // _L


def _body(idx_hbm, emb_hbm, w_hbm, b_hbm, out_hbm,
          idx_v, rows_v, w_v, b_v, out_v, sem):
    c = lax.axis_index("c")
    s = lax.axis_index("s")

    @pl.when(jnp.logical_and(c == 0, s == 0))
    def _():
        pltpu.sync_copy(idx_hbm, idx_v.at[pl.ds(0, _SEQ)])
        copies = []
        for n in range(_SEQ):
            if n % _L == 0:
                chunk = idx_v[pl.ds(n, _L)]
            r = chunk[n % _L]
            copies.append(
                pltpu.async_copy(emb_hbm.at[pl.ds(r, 1)],
                                 rows_v.at[pl.ds(n, 1)], sem))
        pltpu.sync_copy(w_hbm, w_v)
        pltpu.sync_copy(b_hbm, b_v)
        for cp in copies:
            cp.wait()

        init = tuple(rows_v[0, pl.ds(d * _L, _L)] for d in range(_NCH))

        def step(r, accs):
            return tuple(jnp.maximum(a, rows_v[r, pl.ds(d * _L, _L)])
                         for d, a in enumerate(accs))

        accs = lax.fori_loop(1, _SEQ, step, init)

        lane = lax.iota(jnp.int32, _L)
        vec = jnp.zeros((_L,), jnp.float32)
        for j in range(2):
            psum = jnp.zeros((_L,), jnp.float32)
            for d in range(_NCH):
                psum = psum + accs[d] * w_v[pl.ds((j * _D) + d * _L, _L)]
            t = psum[0]
            for i in range(1, _L):
                t = t + psum[i]
            vec = jnp.where(lane == j, t, vec)
        out_v[...] = vec + b_v[...]
        pltpu.sync_copy(out_v, out_hbm)


_mesh = plsc.VectorSubcoreMesh(core_axis_name="c", subcore_axis_name="s",
                               num_cores=2, num_subcores=16)

_call = functools.partial(
    pl.kernel,
    out_type=jax.ShapeDtypeStruct((_L,), jnp.float32),
    mesh=_mesh,
    scratch_types=[
        pltpu.VMEM((_SEQ + _L - _SEQ % _L,), jnp.int32),
        pltpu.VMEM((_SEQ, _D), jnp.float32),
        pltpu.VMEM((2 * _D,), jnp.float32),
        pltpu.VMEM((_L,), jnp.float32),
        pltpu.VMEM((_L,), jnp.float32),
        pltpu.SemaphoreType.DMA,
    ],
)(_body)


@jax.jit
def kernel(x, emb, W, b):
    idx = x.reshape(-1).astype(jnp.int32)
    wflat = W.astype(jnp.float32).reshape(-1)
    b16 = jnp.pad(b.astype(jnp.float32), (0, _L - b.shape[0]))
    out16 = _call(idx, emb, wflat, b16)
    return out16[:2].reshape(1, 2)


# speedup vs baseline: 3.2777x; 1.9236x over previous
import functools

import jax
import jax.numpy as jnp
from jax import lax
from jax.experimental import pallas as pl
from jax.experimental.pallas import tpu as pltpu
from jax.experimental.pallas import tpu_sc as plsc

_L = 16
_D = 64
_SEQ = 200
_NCH = _D // _L
_NT = 16
_K = 16
_NPAD = _NT * _K
_W = 128
_ROWS = 1000000


def _body(idx_hbm, embt_hbm, w_hbm, b_hbm, out_hbm,
          idx_v, bufs_v, acc_v, shared_v, red_v, w_v, b_v, out_v,
          sem0, sem1):
    c = lax.axis_index("c")
    s = lax.axis_index("s")
    sems = (sem0, sem1)

    @pl.when(jnp.logical_and(c == 0, s == 0))
    def _():
        accs = [None] * _NCH
        for blk in range(_NT):
            pltpu.sync_copy(idx_hbm.at[pl.ds(blk * _K, _K)], idx_v)
            chunk = idx_v[...]

            starts = []
            lanes = []
            for i in range(_K):
                r = chunk[i]
                st = (r // _W) * _W
                starts.append(st)
                lanes.append(r - st)

            def issue(i):
                return pltpu.async_copy(
                    embt_hbm.at[:, pl.ds(starts[i], _W)],
                    bufs_v.at[i % 2], sems[i % 2])

            cp = issue(0)
            for i in range(_K):
                nxt = issue(i + 1) if i + 1 < _K else None
                cp.wait()
                buf = bufs_v.at[i % 2]
                col = jnp.full((_L,), lanes[i], jnp.int32)
                for d in range(_NCH):
                    row = lax.iota(jnp.int32, _L) + (d * _L)
                    v = plsc.load_gather(buf, [row, col])
                    accs[d] = v if accs[d] is None else jnp.maximum(accs[d], v)
                cp = nxt

        pltpu.sync_copy(w_hbm, w_v)
        pltpu.sync_copy(b_hbm, b_v.at[pl.ds(0, 2)])
        pooled = accs

        lane = lax.iota(jnp.int32, _L)
        vec = jnp.zeros((_L,), jnp.float32)
        for j in range(2):
            psum = jnp.zeros((_L,), jnp.float32)
            for d in range(_NCH):
                psum = psum + pooled[d] * w_v[pl.ds((j * _D) + d * _L, _L)]
            t = psum[0]
            for i in range(1, _L):
                t = t + psum[i]
            vec = jnp.where(lane == j, t, vec)
        out_v[...] = vec + b_v[...]
        pltpu.sync_copy(out_v, out_hbm)


_mesh = plsc.VectorSubcoreMesh(core_axis_name="c", subcore_axis_name="s",
                               num_cores=2, num_subcores=16)

_call = functools.partial(
    pl.kernel,
    out_type=jax.ShapeDtypeStruct((_L,), jnp.float32),
    mesh=_mesh,
    compiler_params=pltpu.CompilerParams(needs_layout_passes=False),
    scratch_types=[
        pltpu.VMEM((_K,), jnp.int32),
        pltpu.VMEM((2, _D, _W), jnp.float32),
        pltpu.VMEM((_D,), jnp.float32),
        pltpu.VMEM_SHARED((_NT, _D), jnp.float32),
        pltpu.VMEM((_NT, _D), jnp.float32),
        pltpu.VMEM((2 * _D,), jnp.float32),
        pltpu.VMEM((_L,), jnp.float32),
        pltpu.VMEM((_L,), jnp.float32),
        pltpu.SemaphoreType.DMA,
        pltpu.SemaphoreType.DMA,
    ],
)(_body)


@jax.jit
def kernel(x, emb, W, b):
    idx = x.reshape(-1).astype(jnp.int32)
    idx = jnp.concatenate([idx, jnp.full((_NPAD - _SEQ,), idx[0], jnp.int32)])
    wflat = W.astype(jnp.float32).reshape(-1)
    out16 = _call(idx, emb.T, wflat, b.astype(jnp.float32))
    return out16[:2].reshape(1, 2)


# speedup vs baseline: 18.0482x; 5.5063x over previous
import functools

import jax
import jax.numpy as jnp
from jax import lax
from jax.experimental import pallas as pl
from jax.experimental.pallas import tpu as pltpu
from jax.experimental.pallas import tpu_sc as plsc

_L = 16
_D = 64
_SEQ = 200
_NCH = _D // _L
_NW = 32
_KG = 8
_NPAD = _NW * _KG
_W = 128
_SLAB = 128


def _gather_body(idx_hbm, embt_hbm, part_hbm, idx_v, bufs_v, acc_v,
                 sem0, sem1):
    c = lax.axis_index("c")
    s = lax.axis_index("s")
    wid = s * 2 + c
    sems = (sem0, sem1)

    pltpu.sync_copy(idx_hbm.at[pl.ds(wid * _KG, _KG)],
                    idx_v.at[pl.ds(0, _KG)])
    chunk = idx_v[...]

    starts = []
    lanes = []
    for i in range(_KG):
        r = chunk[i]
        st = (r // _W) * _W
        starts.append(st)
        lanes.append(r - st)

    def issue(i):
        return pltpu.async_copy(
            embt_hbm.at[:, pl.ds(starts[i], _W)],
            bufs_v.at[i % 2], sems[i % 2])

    cp = issue(0)
    accs = [None] * _NCH
    for i in range(_KG):
        nxt = issue(i + 1) if i + 1 < _KG else None
        cp.wait()
        buf = bufs_v.at[i % 2]
        col = jnp.full((_L,), lanes[i], jnp.int32)
        for d in range(_NCH):
            row = lax.iota(jnp.int32, _L) + (d * _L)
            v = plsc.load_gather(buf, [row, col])
            accs[d] = v if accs[d] is None else jnp.maximum(accs[d], v)
        cp = nxt

    for d in range(_NCH):
        acc_v[pl.ds(d * _L, _L)] = accs[d]
    pltpu.sync_copy(acc_v, part_hbm.at[pl.ds(wid * _SLAB, _SLAB)])


def _reduce_body(part_hbm, w_hbm, b_hbm, out_hbm, part_v, w_v, b_v, out_v):
    c = lax.axis_index("c")
    s = lax.axis_index("s")

    @pl.when(jnp.logical_and(c == 0, s == 0))
    def _():
        pltpu.sync_copy(part_hbm, part_v)
        pltpu.sync_copy(w_hbm, w_v)
        pltpu.sync_copy(b_hbm, b_v.at[pl.ds(0, 2)])

        pooled = []
        for d in range(_NCH):
            m = part_v[pl.ds(d * _L, _L)]
            for t in range(1, _NW):
                m = jnp.maximum(m, part_v[pl.ds(t * _SLAB + d * _L, _L)])
            pooled.append(m)

        lane = lax.iota(jnp.int32, _L)
        vec = jnp.zeros((_L,), jnp.float32)
        for j in range(2):
            psum = jnp.zeros((_L,), jnp.float32)
            for d in range(_NCH):
                psum = psum + pooled[d] * w_v[pl.ds((j * _D) + d * _L, _L)]
            t = psum[0]
            for i in range(1, _L):
                t = t + psum[i]
            vec = jnp.where(lane == j, t, vec)
        out_v[...] = vec + b_v[...]
        pltpu.sync_copy(out_v, out_hbm)


_mesh = plsc.VectorSubcoreMesh(core_axis_name="c", subcore_axis_name="s",
                               num_cores=2, num_subcores=16)

_gather = functools.partial(
    pl.kernel,
    out_type=jax.ShapeDtypeStruct((_NW * _SLAB,), jnp.float32),
    mesh=_mesh,
    compiler_params=pltpu.CompilerParams(needs_layout_passes=False),
    scratch_types=[
        pltpu.VMEM((_L,), jnp.int32),
        pltpu.VMEM((2, _D, _W), jnp.float32),
        pltpu.VMEM((_SLAB,), jnp.float32),
        pltpu.SemaphoreType.DMA,
        pltpu.SemaphoreType.DMA,
    ],
)(_gather_body)

_reduce = functools.partial(
    pl.kernel,
    out_type=jax.ShapeDtypeStruct((_L,), jnp.float32),
    mesh=_mesh,
    compiler_params=pltpu.CompilerParams(needs_layout_passes=False),
    scratch_types=[
        pltpu.VMEM((_NW * _SLAB,), jnp.float32),
        pltpu.VMEM((2 * _D,), jnp.float32),
        pltpu.VMEM((_L,), jnp.float32),
        pltpu.VMEM((_L,), jnp.float32),
    ],
)(_reduce_body)


@jax.jit
def kernel(x, emb, W, b):
    idx = x.reshape(-1).astype(jnp.int32)
    idx = jnp.concatenate([idx, jnp.full((_NPAD - _SEQ,), idx[0], jnp.int32)])
    wflat = W.astype(jnp.float32).reshape(-1)
    part = _gather(idx, emb.T)
    out16 = _reduce(part, wflat, b.astype(jnp.float32))
    return out16[:2].reshape(1, 2)


# speedup vs baseline: 20.0784x; 1.1125x over previous
import functools

import jax
import jax.numpy as jnp
from jax import lax
from jax.experimental import pallas as pl
from jax.experimental.pallas import tpu as pltpu
from jax.experimental.pallas import tpu_sc as plsc

_L = 16
_D = 64
_SEQ = 200
_NCH = _D // _L
_NW = 32
_KG = 8
_NPAD = _NW * _KG
_W = 128
_SLAB = 128
_NBUF = 4


def _gather_body(idx_hbm, embt_hbm, part_hbm, idx_v, bufs_v, acc_v,
                 sem0, sem1, sem2, sem3):
    c = lax.axis_index("c")
    s = lax.axis_index("s")
    wid = s * 2 + c
    sems = (sem0, sem1, sem2, sem3)

    off = jnp.minimum(wid * _KG, _SEQ - _KG)
    pltpu.sync_copy(idx_hbm.at[pl.ds(off, _KG)],
                    idx_v.at[pl.ds(0, _KG)])
    chunk = idx_v[...]

    starts = []
    lanes = []
    for i in range(_KG):
        r = chunk[i]
        st = (r // _W) * _W
        starts.append(st)
        lanes.append(r - st)

    def issue(i):
        return pltpu.async_copy(
            embt_hbm.at[:, pl.ds(starts[i], _W)],
            bufs_v.at[i % _NBUF], sems[i % _NBUF])

    cps = [issue(i) for i in range(_NBUF - 1)]
    accs = [None] * _NCH
    for i in range(_KG):
        if i + _NBUF - 1 < _KG:
            cps.append(issue(i + _NBUF - 1))
        cps[i].wait()
        buf = bufs_v.at[i % _NBUF]
        col = jnp.full((_L,), lanes[i], jnp.int32)
        for d in range(_NCH):
            row = lax.iota(jnp.int32, _L) + (d * _L)
            v = plsc.load_gather(buf, [row, col])
            accs[d] = v if accs[d] is None else jnp.maximum(accs[d], v)

    for d in range(_NCH):
        acc_v[pl.ds(d * _L, _L)] = accs[d]
    pltpu.sync_copy(acc_v, part_hbm.at[pl.ds(wid * _SLAB, _SLAB)])


def _reduce_body(part_hbm, w_hbm, b_hbm, out_hbm, part_v, w_v, b_v, out_v):
    c = lax.axis_index("c")
    s = lax.axis_index("s")

    @pl.when(jnp.logical_and(c == 0, s == 0))
    def _():
        pltpu.sync_copy(part_hbm, part_v)
        pltpu.sync_copy(w_hbm, w_v)
        pltpu.sync_copy(b_hbm, b_v.at[pl.ds(0, 2)])

        pooled = []
        for d in range(_NCH):
            m = part_v[pl.ds(d * _L, _L)]
            for t in range(1, _NW):
                m = jnp.maximum(m, part_v[pl.ds(t * _SLAB + d * _L, _L)])
            pooled.append(m)

        lane = lax.iota(jnp.int32, _L)
        vec = jnp.zeros((_L,), jnp.float32)
        for j in range(2):
            psum = jnp.zeros((_L,), jnp.float32)
            for d in range(_NCH):
                psum = psum + pooled[d] * w_v[j, pl.ds(d * _L, _L)]
            t = psum[0]
            for i in range(1, _L):
                t = t + psum[i]
            vec = jnp.where(lane == j, t, vec)
        out_v[...] = vec + b_v[...]
        pltpu.sync_copy(out_v.at[pl.ds(0, 2)], out_hbm.at[0])


_mesh = plsc.VectorSubcoreMesh(core_axis_name="c", subcore_axis_name="s",
                               num_cores=2, num_subcores=16)

_gather = functools.partial(
    pl.kernel,
    out_type=jax.ShapeDtypeStruct((_NW * _SLAB,), jnp.float32),
    mesh=_mesh,
    compiler_params=pltpu.CompilerParams(needs_layout_passes=False),
    scratch_types=[
        pltpu.VMEM((_L,), jnp.int32),
        pltpu.VMEM((_NBUF, _D, _W), jnp.float32),
        pltpu.VMEM((_SLAB,), jnp.float32),
        pltpu.SemaphoreType.DMA,
        pltpu.SemaphoreType.DMA,
        pltpu.SemaphoreType.DMA,
        pltpu.SemaphoreType.DMA,
    ],
)(_gather_body)

_reduce = functools.partial(
    pl.kernel,
    out_type=jax.ShapeDtypeStruct((1, 2), jnp.float32),
    mesh=_mesh,
    compiler_params=pltpu.CompilerParams(needs_layout_passes=False),
    scratch_types=[
        pltpu.VMEM((_NW * _SLAB,), jnp.float32),
        pltpu.VMEM((2, _D), jnp.float32),
        pltpu.VMEM((_L,), jnp.float32),
        pltpu.VMEM((_L,), jnp.float32),
    ],
)(_reduce_body)


@jax.jit
def kernel(x, emb, W, b):
    idx = x.reshape(-1).astype(jnp.int32)
    part = _gather(idx, emb.T)
    return _reduce(part, W.astype(jnp.float32), b.astype(jnp.float32))


# speedup vs baseline: 20.2277x; 1.0074x over previous
import functools

import jax
import jax.numpy as jnp
from jax import lax
from jax.experimental import pallas as pl
from jax.experimental.pallas import tpu as pltpu
from jax.experimental.pallas import tpu_sc as plsc

_L = 16
_D = 64
_SEQ = 200
_NCH = _D // _L
_NW = 32
_KG = 8
_NPAD = _NW * _KG
_W = 128
_SLAB = 128
_NBUF = 4


def _gather_body(idx_hbm, embt_hbm, part_hbm, idx_v, bufs_v, acc_v,
                 sem0, sem1, sem2, sem3):
    c = lax.axis_index("c")
    s = lax.axis_index("s")
    wid = s * 2 + c
    sems = (sem0, sem1, sem2, sem3)

    off = jnp.minimum(wid * _KG, _SEQ - _KG)
    pltpu.sync_copy(idx_hbm.at[pl.ds(off, _KG)],
                    idx_v.at[pl.ds(0, _KG)])
    chunk = idx_v[...]

    starts = []
    lanes = []
    for i in range(_KG):
        r = chunk[i]
        st = (r // _W) * _W
        starts.append(st)
        lanes.append(r - st)

    def issue(i):
        return pltpu.async_copy(
            embt_hbm.at[:, pl.ds(starts[i], _W)],
            bufs_v.at[i % _NBUF], sems[i % _NBUF])

    cps = [issue(i) for i in range(_NBUF - 1)]
    accs = [None] * _NCH
    for i in range(_KG):
        if i + _NBUF - 1 < _KG:
            cps.append(issue(i + _NBUF - 1))
        cps[i].wait()
        buf = bufs_v.at[i % _NBUF]
        col = jnp.full((_L,), lanes[i], jnp.int32)
        for d in range(_NCH):
            row = lax.iota(jnp.int32, _L) + (d * _L)
            v = plsc.load_gather(buf, [row, col])
            accs[d] = v if accs[d] is None else jnp.maximum(accs[d], v)

    for d in range(_NCH):
        acc_v[pl.ds(d * _L, _L)] = accs[d]
    pltpu.sync_copy(acc_v, part_hbm.at[pl.ds(wid * _SLAB, _SLAB)])


def _reduce_body(part_hbm, w_hbm, b_hbm, out_hbm, part_v, w_v, b_v, out_v):
    c = lax.axis_index("c")
    s = lax.axis_index("s")

    @pl.when(jnp.logical_and(c == 0, s == 0))
    def _():
        pltpu.sync_copy(part_hbm, part_v)
        pltpu.sync_copy(w_hbm, w_v)
        pltpu.sync_copy(b_hbm, b_v.at[pl.ds(0, 2)])

        pooled = []
        for d in range(_NCH):
            m = part_v[pl.ds(d * _L, _L)]
            for t in range(1, _NW):
                m = jnp.maximum(m, part_v[pl.ds(t * _SLAB + d * _L, _L)])
            pooled.append(m)

        lane = lax.iota(jnp.int32, _L)
        vec = jnp.zeros((_L,), jnp.float32)
        for j in range(2):
            psum = jnp.zeros((_L,), jnp.float32)
            for d in range(_NCH):
                psum = psum + pooled[d] * w_v[j, pl.ds(d * _L, _L)]
            t = psum[0]
            for i in range(1, _L):
                t = t + psum[i]
            vec = jnp.where(lane == j, t, vec)
        out_v[...] = vec + b_v[...]
        pltpu.sync_copy(out_v.at[pl.ds(0, 2)], out_hbm.at[0])


_mesh = plsc.VectorSubcoreMesh(core_axis_name="c", subcore_axis_name="s",
                               num_cores=2, num_subcores=16)

_gather = functools.partial(
    pl.kernel,
    out_type=jax.ShapeDtypeStruct((_NW * _SLAB,), jnp.float32),
    mesh=_mesh,
    compiler_params=pltpu.CompilerParams(needs_layout_passes=False, skip_device_barrier=True),
    scratch_types=[
        pltpu.VMEM((_L,), jnp.int32),
        pltpu.VMEM((_NBUF, _D, _W), jnp.float32),
        pltpu.VMEM((_SLAB,), jnp.float32),
        pltpu.SemaphoreType.DMA,
        pltpu.SemaphoreType.DMA,
        pltpu.SemaphoreType.DMA,
        pltpu.SemaphoreType.DMA,
    ],
)(_gather_body)

_reduce = functools.partial(
    pl.kernel,
    out_type=jax.ShapeDtypeStruct((1, 2), jnp.float32),
    mesh=_mesh,
    compiler_params=pltpu.CompilerParams(needs_layout_passes=False, skip_device_barrier=True),
    scratch_types=[
        pltpu.VMEM((_NW * _SLAB,), jnp.float32),
        pltpu.VMEM((2, _D), jnp.float32),
        pltpu.VMEM((_L,), jnp.float32),
        pltpu.VMEM((_L,), jnp.float32),
    ],
)(_reduce_body)


@jax.jit
def kernel(x, emb, W, b):
    idx = x.reshape(-1).astype(jnp.int32)
    part = _gather(idx, emb.T)
    return _reduce(part, W.astype(jnp.float32), b.astype(jnp.float32))


# speedup vs baseline: 20.7972x; 1.0282x over previous
import functools

import jax
import jax.numpy as jnp
from jax import lax
from jax.experimental import pallas as pl
from jax.experimental.pallas import tpu as pltpu
from jax.experimental.pallas import tpu_sc as plsc

_L = 16
_D = 64
_SEQ = 200
_NCH = _D // _L
_NW = 32
_NACT = 25
_KG = 8
_W = 128
_SLAB = 128
_NBUF = 4


def _gather_body(idx_hbm, embt_hbm, part_hbm, idx_v, bufs_v, acc_v,
                 sem0, sem1, sem2, sem3):
    c = lax.axis_index("c")
    s = lax.axis_index("s")
    wid = s * 2 + c
    sems = (sem0, sem1, sem2, sem3)

    @pl.when(wid < _NACT)
    def _():
        _gather_tile(idx_hbm, embt_hbm, part_hbm, idx_v, bufs_v, acc_v,
                     sems, wid)


def _gather_tile(idx_hbm, embt_hbm, part_hbm, idx_v, bufs_v, acc_v,
                 sems, wid):
    pltpu.sync_copy(idx_hbm.at[pl.ds(wid * _KG, _KG)],
                    idx_v.at[pl.ds(0, _KG)])
    chunk = idx_v[...]

    starts = []
    lanes = []
    for i in range(_KG):
        r = chunk[i]
        st = (r // _W) * _W
        starts.append(st)
        lanes.append(r - st)

    def issue(i):
        return pltpu.async_copy(
            embt_hbm.at[:, pl.ds(starts[i], _W)],
            bufs_v.at[i % _NBUF], sems[i % _NBUF])

    cps = [issue(i) for i in range(_NBUF - 1)]
    accs = [None] * _NCH
    for i in range(_KG):
        if i + _NBUF - 1 < _KG:
            cps.append(issue(i + _NBUF - 1))
        cps[i].wait()
        buf = bufs_v.at[i % _NBUF]
        col = jnp.full((_L,), lanes[i], jnp.int32)
        for d in range(_NCH):
            row = lax.iota(jnp.int32, _L) + (d * _L)
            v = plsc.load_gather(buf, [row, col])
            accs[d] = v if accs[d] is None else jnp.maximum(accs[d], v)

    for d in range(_NCH):
        acc_v[pl.ds(d * _L, _L)] = accs[d]
    pltpu.sync_copy(acc_v, part_hbm.at[pl.ds(wid * _SLAB, _SLAB)])


def _reduce_body(part_hbm, w_hbm, b_hbm, out_hbm, part_v, w_v, b_v, out_v):
    c = lax.axis_index("c")
    s = lax.axis_index("s")

    @pl.when(jnp.logical_and(c == 0, s == 0))
    def _():
        pltpu.sync_copy(part_hbm, part_v)
        pltpu.sync_copy(w_hbm, w_v)
        pltpu.sync_copy(b_hbm, b_v.at[pl.ds(0, 2)])

        pooled = []
        for d in range(_NCH):
            m = part_v[pl.ds(d * _L, _L)]
            for t in range(1, _NACT):
                m = jnp.maximum(m, part_v[pl.ds(t * _SLAB + d * _L, _L)])
            pooled.append(m)

        lane = lax.iota(jnp.int32, _L)
        vec = jnp.zeros((_L,), jnp.float32)
        for j in range(2):
            psum = jnp.zeros((_L,), jnp.float32)
            for d in range(_NCH):
                psum = psum + pooled[d] * w_v[j, pl.ds(d * _L, _L)]
            t = psum[0]
            for i in range(1, _L):
                t = t + psum[i]
            vec = jnp.where(lane == j, t, vec)
        out_v[...] = vec + b_v[...]
        pltpu.sync_copy(out_v.at[pl.ds(0, 2)], out_hbm.at[0])


_mesh = plsc.VectorSubcoreMesh(core_axis_name="c", subcore_axis_name="s",
                               num_cores=2, num_subcores=16)

_gather = functools.partial(
    pl.kernel,
    out_type=jax.ShapeDtypeStruct((_NACT * _SLAB,), jnp.float32),
    mesh=_mesh,
    compiler_params=pltpu.CompilerParams(needs_layout_passes=False, skip_device_barrier=True),
    scratch_types=[
        pltpu.VMEM((_L,), jnp.int32),
        pltpu.VMEM((_NBUF, _D, _W), jnp.float32),
        pltpu.VMEM((_SLAB,), jnp.float32),
        pltpu.SemaphoreType.DMA,
        pltpu.SemaphoreType.DMA,
        pltpu.SemaphoreType.DMA,
        pltpu.SemaphoreType.DMA,
    ],
)(_gather_body)

_reduce = functools.partial(
    pl.kernel,
    out_type=jax.ShapeDtypeStruct((1, 2), jnp.float32),
    mesh=_mesh,
    compiler_params=pltpu.CompilerParams(needs_layout_passes=False, skip_device_barrier=True),
    scratch_types=[
        pltpu.VMEM((_NACT * _SLAB,), jnp.float32),
        pltpu.VMEM((2, _D), jnp.float32),
        pltpu.VMEM((_L,), jnp.float32),
        pltpu.VMEM((_L,), jnp.float32),
    ],
)(_reduce_body)


@jax.jit
def kernel(x, emb, W, b):
    idx = x.reshape(-1).astype(jnp.int32)
    part = _gather(idx, emb.T)
    return _reduce(part, W.astype(jnp.float32), b.astype(jnp.float32))
